# baseline (speedup 1.0000x reference)
import jax
import jax.numpy as jnp
from jax import lax
from jax.experimental import pallas as pl
from jax.experimental.pallas import tpu as pltpu
from jax.experimental.pallas import tpu_sc as plsc

NC = 2
NS = 16
CHUNK = 128
K_GROUP = 16

D = 128
N_PAD = 10240
ROWS_PER_TILE = N_PAD // NS
ZCHUNKS = ROWS_PER_TILE // CHUNK


def _sc_aggregate(g_core0, g_core1):
  mesh = plsc.VectorSubcoreMesh(core_axis_name="c", subcore_axis_name="s")

  def body(x_hbm, seg_hbm, gid_hbm, zrow_hbm, ones_hbm,
           out_sum, out_cnt,
           seg_v, gid_v, rows_a, rows_b, sem_g, sem_s, acc):
    cid = lax.axis_index("c")
    sid = lax.axis_index("s")
    row_lo = sid * ROWS_PER_TILE
    n_groups = jnp.where(cid == 0, g_core0, g_core1)

    def zero_acc():
      pltpu.sync_copy(zrow_hbm, rows_a)
      for i in range(ZCHUNKS):
        pltpu.async_copy(rows_a, acc.at[pl.ds(row_lo + i * CHUNK, CHUNK)],
                         sem_s)
      for i in range(ZCHUNKS):
        pltpu.make_async_copy(
            rows_a, acc.at[pl.ds(row_lo, CHUNK)], sem_s).wait()

    def writeback(out_hbm):
      pltpu.async_copy(acc.at[pl.ds(row_lo, CHUNK)], rows_a, sem_g)
      for i in range(ZCHUNKS):
        buf = rows_a if i % 2 == 0 else rows_b
        nxt = rows_b if i % 2 == 0 else rows_a
        pltpu.make_async_copy(
            acc.at[pl.ds(row_lo, CHUNK)], buf, sem_g).wait()
        if i + 1 < ZCHUNKS:
          pltpu.async_copy(
              acc.at[pl.ds(row_lo + (i + 1) * CHUNK, CHUNK)], nxt, sem_g)
        pltpu.sync_copy(buf, out_hbm.at[cid, pl.ds(row_lo + i * CHUNK,
                                                   CHUNK)])

    zero_acc()
    plsc.subcore_barrier()

    def group_sum(g, carry):
      pltpu.sync_copy(seg_hbm.at[cid, sid, pl.ds(g * K_GROUP, K_GROUP)],
                      seg_v)
      pltpu.sync_copy(gid_hbm.at[cid, sid, pl.ds(g * K_GROUP, K_GROUP)],
                      gid_v)
      pltpu.async_copy(x_hbm.at[gid_v.at[0]], rows_a, sem_g)
      for k in range(K_GROUP):
        buf = rows_a if k % 2 == 0 else rows_b
        nxt = rows_b if k % 2 == 0 else rows_a
        pltpu.make_async_copy(x_hbm.at[gid_v.at[k]], buf, sem_g).wait()
        pltpu.async_copy(buf, acc.at[seg_v.at[k]], sem_s, add=True)
        if k >= 1:
          pltpu.make_async_copy(nxt, acc.at[seg_v.at[k]], sem_s).wait()
        if k + 1 < K_GROUP:
          pltpu.async_copy(x_hbm.at[gid_v.at[k + 1]], nxt, sem_g)
      pltpu.make_async_copy(rows_a, acc.at[seg_v.at[0]], sem_s).wait()
      return carry

    lax.fori_loop(0, n_groups, group_sum, 0)
    plsc.subcore_barrier()
    writeback(out_sum)

    zero_acc()
    plsc.subcore_barrier()
    pltpu.sync_copy(ones_hbm, rows_a)

    def group_cnt(g, carry):
      pltpu.sync_copy(seg_hbm.at[cid, sid, pl.ds(g * K_GROUP, K_GROUP)],
                      seg_v)
      for k in range(K_GROUP):
        pltpu.async_copy(rows_a, acc.at[seg_v.at[k]], sem_s, add=True)
      for k in range(K_GROUP):
        pltpu.make_async_copy(rows_a, acc.at[seg_v.at[0]], sem_s).wait()
      return carry

    lax.fori_loop(0, n_groups, group_cnt, 0)
    plsc.subcore_barrier()
    writeback(out_cnt)

  return pl.kernel(
      body,
      out_type=(
          jax.ShapeDtypeStruct((NC, N_PAD, D), jnp.float32),
          jax.ShapeDtypeStruct((NC, N_PAD, D), jnp.float32),
      ),
      mesh=mesh,
      scratch_types=[
          pltpu.VMEM((K_GROUP, CHUNK), jnp.int32),
          pltpu.VMEM((K_GROUP, CHUNK), jnp.int32),
          pltpu.VMEM((CHUNK, D), jnp.float32),
          pltpu.VMEM((CHUNK, D), jnp.float32),
          pltpu.SemaphoreType.DMA,
          pltpu.SemaphoreType.DMA,
          pltpu.VMEM_SHARED((N_PAD, D), jnp.float32),
      ],
  )


def _tc_body(x_ref, p0_ref, p1_ref, c0_ref, c1_ref, ws_ref, wn_ref, o_ref):
  x = x_ref[...]
  sums = p0_ref[0] + p1_ref[0]
  counts = c0_ref[0][:, 0:1] + c1_ref[0][:, 0:1]
  mean = sums / jnp.maximum(counts, 1.0)
  h = jnp.dot(mean, wn_ref[...], preferred_element_type=jnp.float32)
  h = jnp.where(h >= 0, h, 0.2 * h)
  s = jnp.dot(x, ws_ref[...], preferred_element_type=jnp.float32)
  u = s + h
  u = jnp.where(u >= 0, u, 0.2 * u)
  nrm = jnp.sqrt(jnp.sum(u * u, axis=1, keepdims=True))
  o_ref[...] = u / jnp.maximum(nrm, 1e-12)


def kernel(node_fts, edge_fts, edges, W_self, W_neigh):
  del edge_fts
  n = node_fts.shape[0]
  e2 = 2 * edges.shape[1]

  per_tile_group = CHUNK * K_GROUP
  total_groups = -(-e2 // (NS * per_tile_group))
  g0 = max(1, round(total_groups * 12 / 20))
  g1 = -(-(e2 - g0 * NS * per_tile_group) // (NS * per_tile_group))
  e_pad = (g0 + g1) * NS * per_tile_group
  n_chunks = max(g0, g1) * K_GROUP

  seg = jnp.concatenate([edges[0], edges[1]])
  gid = jnp.concatenate([edges[1], edges[0]])
  pad = e_pad - e2
  seg = jnp.concatenate([seg, jnp.full((pad,), N_PAD - 8, jnp.int32)])
  gid = jnp.concatenate([gid, jnp.zeros((pad,), jnp.int32)])

  def split_core(a):
    cut = g0 * NS * per_tile_group
    a0 = a[:cut].reshape(NS, g0 * K_GROUP, CHUNK)
    a1 = a[cut:].reshape(NS, g1 * K_GROUP, CHUNK)
    if g0 < g1:
      a0 = jnp.pad(a0, ((0, 0), (0, (g1 - g0) * K_GROUP), (0, 0)),
                   constant_values=N_PAD - 8)
    elif g1 < g0:
      a1 = jnp.pad(a1, ((0, 0), (0, (g0 - g1) * K_GROUP), (0, 0)),
                   constant_values=N_PAD - 8)
    return jnp.stack([a0, a1])

  seg = split_core(seg)
  gid = split_core(gid)

  zrow = jnp.zeros((CHUNK, D), jnp.float32)
  ones = jnp.ones((CHUNK, D), jnp.float32)

  part_sum, part_cnt = _sc_aggregate(g0, g1)(
      node_fts, seg, gid, zrow, ones)

  bn = 1000
  grid = n // bn
  out = pl.pallas_call(
      _tc_body,
      grid=(grid,),
      in_specs=[
          pl.BlockSpec((bn, D), lambda i: (i, 0)),
          pl.BlockSpec((1, bn, D), lambda i: (0, i, 0)),
          pl.BlockSpec((1, bn, D), lambda i: (1, i, 0)),
          pl.BlockSpec((1, bn, D), lambda i: (0, i, 0)),
          pl.BlockSpec((1, bn, D), lambda i: (1, i, 0)),
          pl.BlockSpec((D, D), lambda i: (0, 0)),
          pl.BlockSpec((D, D), lambda i: (0, 0)),
      ],
      out_specs=pl.BlockSpec((bn, D), lambda i: (i, 0)),
      out_shape=jax.ShapeDtypeStruct((n, D), jnp.float32),
  )(node_fts, part_sum, part_sum, part_cnt, part_cnt,
    W_self.T, W_neigh.T)
  return out


# speedup vs baseline: 1.0460x; 1.0460x over previous
import jax
import jax.numpy as jnp
from jax import lax
from jax.experimental import pallas as pl
from jax.experimental.pallas import tpu as pltpu
from jax.experimental.pallas import tpu_sc as plsc

NC = 2
NS = 16
CHUNK = 128
K_GROUP = 16

D = 128
N_PAD = 10240
ROWS_PER_TILE = N_PAD // NS
ZCHUNKS = ROWS_PER_TILE // CHUNK


def _sc_aggregate(g_core0, g_core1):
  mesh = plsc.VectorSubcoreMesh(core_axis_name="c", subcore_axis_name="s")

  def body(x_hbm, seg_hbm, gid_hbm, zrow_hbm, ones_hbm,
           out_sum, out_cnt,
           seg_v, gid_v, rows_a, rows_b, sem_g, sem_s, acc):
    cid = lax.axis_index("c")
    sid = lax.axis_index("s")
    row_lo = sid * ROWS_PER_TILE
    n_groups = jnp.where(cid == 0, g_core0, g_core1)

    def zero_acc():
      pltpu.sync_copy(zrow_hbm, rows_a)
      for i in range(ZCHUNKS):
        pltpu.async_copy(rows_a, acc.at[pl.ds(row_lo + i * CHUNK, CHUNK)],
                         sem_s)
      for i in range(ZCHUNKS):
        pltpu.make_async_copy(
            rows_a, acc.at[pl.ds(row_lo, CHUNK)], sem_s).wait()

    def writeback(out_hbm):
      pltpu.async_copy(acc.at[pl.ds(row_lo, CHUNK)], rows_a, sem_g)
      for i in range(ZCHUNKS):
        buf = rows_a if i % 2 == 0 else rows_b
        nxt = rows_b if i % 2 == 0 else rows_a
        pltpu.make_async_copy(
            acc.at[pl.ds(row_lo, CHUNK)], buf, sem_g).wait()
        if i + 1 < ZCHUNKS:
          pltpu.async_copy(
              acc.at[pl.ds(row_lo + (i + 1) * CHUNK, CHUNK)], nxt, sem_g)
        pltpu.sync_copy(buf, out_hbm.at[cid, pl.ds(row_lo + i * CHUNK,
                                                   CHUNK)])

    zero_acc()
    plsc.subcore_barrier()

    def group_sum(g, carry):
      pltpu.sync_copy(seg_hbm.at[cid, sid, pl.ds(g * K_GROUP, K_GROUP)],
                      seg_v)
      pltpu.sync_copy(gid_hbm.at[cid, sid, pl.ds(g * K_GROUP, K_GROUP)],
                      gid_v)
      pltpu.async_copy(x_hbm.at[gid_v.at[0]], rows_a, sem_g)
      for k in range(K_GROUP):
        buf = rows_a if k % 2 == 0 else rows_b
        nxt = rows_b if k % 2 == 0 else rows_a
        pltpu.make_async_copy(x_hbm.at[gid_v.at[k]], buf, sem_g).wait()
        pltpu.async_copy(buf, acc.at[seg_v.at[k]], sem_s, add=True)
        if k >= 1:
          pltpu.make_async_copy(nxt, acc.at[seg_v.at[k]], sem_s).wait()
        if k + 1 < K_GROUP:
          pltpu.async_copy(x_hbm.at[gid_v.at[k + 1]], nxt, sem_g)
      pltpu.make_async_copy(rows_a, acc.at[seg_v.at[0]], sem_s).wait()
      return carry

    lax.fori_loop(0, n_groups, group_sum, 0)
    plsc.subcore_barrier()
    writeback(out_sum)

    zero_acc()
    plsc.subcore_barrier()
    pltpu.sync_copy(ones_hbm, rows_a)

    def group_cnt(g, carry):
      pltpu.sync_copy(seg_hbm.at[cid, sid, pl.ds(g * K_GROUP, K_GROUP)],
                      seg_v)
      for k in range(K_GROUP):
        pltpu.async_copy(rows_a, acc.at[seg_v.at[k]], sem_s, add=True)
      for k in range(K_GROUP):
        pltpu.make_async_copy(rows_a, acc.at[seg_v.at[0]], sem_s).wait()
      return carry

    lax.fori_loop(0, n_groups, group_cnt, 0)
    plsc.subcore_barrier()
    writeback(out_cnt)

  return pl.kernel(
      body,
      out_type=(
          jax.ShapeDtypeStruct((NC, N_PAD, D), jnp.float32),
          jax.ShapeDtypeStruct((NC, N_PAD, D), jnp.float32),
      ),
      mesh=mesh,
      scratch_types=[
          pltpu.VMEM((K_GROUP, CHUNK), jnp.int32),
          pltpu.VMEM((K_GROUP, CHUNK), jnp.int32),
          pltpu.VMEM((CHUNK, D), jnp.float32),
          pltpu.VMEM((CHUNK, D), jnp.float32),
          pltpu.SemaphoreType.DMA,
          pltpu.SemaphoreType.DMA,
          pltpu.VMEM_SHARED((N_PAD, D), jnp.float32),
      ],
  )


def _tc_body(x_ref, p0_ref, p1_ref, c0_ref, c1_ref, ws_ref, wn_ref, o_ref):
  x = x_ref[...]
  sums = p0_ref[0] + p1_ref[0]
  counts = c0_ref[0][:, 0:1] + c1_ref[0][:, 0:1]
  mean = sums / jnp.maximum(counts, 1.0)
  h = jnp.dot(mean, wn_ref[...], preferred_element_type=jnp.float32)
  h = jnp.where(h >= 0, h, 0.2 * h)
  s = jnp.dot(x, ws_ref[...], preferred_element_type=jnp.float32)
  u = s + h
  u = jnp.where(u >= 0, u, 0.2 * u)
  nrm = jnp.sqrt(jnp.sum(u * u, axis=1, keepdims=True))
  o_ref[...] = u / jnp.maximum(nrm, 1e-12)


def kernel(node_fts, edge_fts, edges, W_self, W_neigh):
  del edge_fts
  n = node_fts.shape[0]
  e2 = 2 * edges.shape[1]

  per_tile_group = CHUNK * K_GROUP
  total_groups = -(-e2 // (NS * per_tile_group))
  g0 = max(1, round(total_groups * 14 / 20))
  g1 = -(-(e2 - g0 * NS * per_tile_group) // (NS * per_tile_group))
  e_pad = (g0 + g1) * NS * per_tile_group
  n_chunks = max(g0, g1) * K_GROUP

  seg = jnp.concatenate([edges[0], edges[1]])
  gid = jnp.concatenate([edges[1], edges[0]])
  pad = e_pad - e2
  seg = jnp.concatenate([seg, jnp.full((pad,), N_PAD - 8, jnp.int32)])
  gid = jnp.concatenate([gid, jnp.zeros((pad,), jnp.int32)])

  def split_core(a):
    cut = g0 * NS * per_tile_group
    a0 = a[:cut].reshape(NS, g0 * K_GROUP, CHUNK)
    a1 = a[cut:].reshape(NS, g1 * K_GROUP, CHUNK)
    if g0 < g1:
      a0 = jnp.pad(a0, ((0, 0), (0, (g1 - g0) * K_GROUP), (0, 0)),
                   constant_values=N_PAD - 8)
    elif g1 < g0:
      a1 = jnp.pad(a1, ((0, 0), (0, (g0 - g1) * K_GROUP), (0, 0)),
                   constant_values=N_PAD - 8)
    return jnp.stack([a0, a1])

  seg = split_core(seg)
  gid = split_core(gid)

  zrow = jnp.zeros((CHUNK, D), jnp.float32)
  ones = jnp.ones((CHUNK, D), jnp.float32)

  part_sum, part_cnt = _sc_aggregate(g0, g1)(
      node_fts, seg, gid, zrow, ones)

  bn = 1000
  grid = n // bn
  out = pl.pallas_call(
      _tc_body,
      grid=(grid,),
      in_specs=[
          pl.BlockSpec((bn, D), lambda i: (i, 0)),
          pl.BlockSpec((1, bn, D), lambda i: (0, i, 0)),
          pl.BlockSpec((1, bn, D), lambda i: (1, i, 0)),
          pl.BlockSpec((1, bn, D), lambda i: (0, i, 0)),
          pl.BlockSpec((1, bn, D), lambda i: (1, i, 0)),
          pl.BlockSpec((D, D), lambda i: (0, 0)),
          pl.BlockSpec((D, D), lambda i: (0, 0)),
      ],
      out_specs=pl.BlockSpec((bn, D), lambda i: (i, 0)),
      out_shape=jax.ShapeDtypeStruct((n, D), jnp.float32),
  )(node_fts, part_sum, part_sum, part_cnt, part_cnt,
    W_self.T, W_neigh.T)
  return out
